# baseline (speedup 1.0000x reference)
import functools

import jax
import jax.numpy as jnp
from jax import lax
from jax.experimental import pallas as pl
from jax.experimental.pallas import tpu as pltpu
from jax.experimental.pallas import tpu_sc as plsc

N, E, D, H, C = 10000, 320000, 128, 128, 40

NC = 2
NS = 16
NW = NC * NS
K = 128
B_CORE = (120, 40)
TOT_BLOCKS = NS * (B_CORE[0] + B_CORE[1])
E_PAD = TOT_BLOCKS * K
CH = 8
MAX_CHUNKS = B_CORE[0] // CH
ROWS_PER_TILE = 624


def _splat(wvec, l):
    return lax.gather(
        wvec, jnp.full((16, 1), l, jnp.int32),
        lax.GatherDimensionNumbers(
            offset_dims=(), collapsed_slice_dims=(0,), start_index_map=(0,)),
        slice_sizes=(1,),
        mode=lax.GatherScatterMode.PROMISE_IN_BOUNDS)


def _agg_body(y_hbm, src_hbm, dst_hbm, w_hbm, out_hbm,
              src_v, dst_v, w_v, rows_v, scaled_v, acc_sh, sem):
    c = lax.axis_index("c")
    s = lax.axis_index("s")
    gbase = jnp.where(c == 0, s * B_CORE[0],
                      NS * B_CORE[0] + s * B_CORE[1])
    nch = jnp.where(c == 0, B_CORE[0] // CH, B_CORE[1] // CH)

    def zero_body(i, _):
        r = i // 8
        j = i % 8
        scaled_v[r, pl.ds(j * 16, 16)] = jnp.zeros((16,), jnp.float32)
        return 0
    lax.fori_loop(0, K * 8, zero_body, 0)
    base_r = s * ROWS_PER_TILE
    for off in (0, 128, 256, 384, ROWS_PER_TILE - K):
        pltpu.sync_copy(scaled_v, acc_sh.at[pl.ds(base_r + off, K)])

    @pl.when(s == NS - 1)
    def _zero_tail():
        pltpu.sync_copy(scaled_v, acc_sh.at[pl.ds(N - K, K)])
    plsc.subcore_barrier()

    def start_gather(i, buf):
        pltpu.async_copy(y_hbm.at[src_v.at[i]], rows_v.at[buf], sem)

    def wait_gather(buf):
        pltpu.make_async_copy(y_hbm.at[pl.ds(0, K)], rows_v.at[buf],
                              sem).wait()

    def process(i, buf):
        def scale_body(g, _):
            wvec = w_v[i, pl.ds(g * 16, 16)]
            for l in range(16):
                k = g * 16 + l
                wk = _splat(wvec, l)
                for j in range(4):
                    vi = rows_v[buf, k, pl.ds(j * 16, 16)]
                    a = lax.bitcast_convert_type(vi << 16, jnp.float32)
                    b = lax.bitcast_convert_type(
                        vi & jnp.int32(-65536), jnp.float32)
                    scaled_v[k, pl.ds(j * 32, 16)] = a * wk
                    scaled_v[k, pl.ds(j * 32 + 16, 16)] = b * wk
            return 0
        lax.fori_loop(0, K // 16, scale_body, 0)
        pltpu.sync_copy(scaled_v, acc_sh.at[dst_v.at[i]], add=True)

    def chunk_body(ch, _):
        @pl.when(ch < nch)
        def _do_chunk():
            row0 = gbase + ch * CH
            pltpu.sync_copy(src_hbm.at[pl.ds(row0, CH)], src_v)
            pltpu.sync_copy(dst_hbm.at[pl.ds(row0, CH)], dst_v)
            pltpu.sync_copy(w_hbm.at[pl.ds(row0, CH)], w_v)
            start_gather(0, 0)

            def block_body(g, _):
                for b in range(2):
                    i = g * 2 + b
                    wait_gather(b)

                    @pl.when(i + 1 < CH)
                    def _next():
                        start_gather(i + 1, 1 - b)
                    process(i, b)
                return 0
            lax.fori_loop(0, CH // 2, block_body, 0)
        return 0
    lax.fori_loop(0, MAX_CHUNKS, chunk_body, 0)

    plsc.subcore_barrier()
    pltpu.sync_copy(acc_sh.at[pl.ds(base_r, ROWS_PER_TILE)],
                    out_hbm.at[c].at[pl.ds(base_r, ROWS_PER_TILE)])

    @pl.when(s == NS - 1)
    def _copy_tail():
        tail = N - NS * ROWS_PER_TILE
        pltpu.sync_copy(acc_sh.at[pl.ds(NS * ROWS_PER_TILE, tail)],
                        out_hbm.at[c].at[pl.ds(NS * ROWS_PER_TILE, tail)])


@functools.lru_cache(maxsize=1)
def _make_agg():
    return pl.kernel(
        _agg_body,
        out_type=jax.ShapeDtypeStruct((NC, N, H), jnp.float32),
        mesh=plsc.VectorSubcoreMesh(core_axis_name="c", subcore_axis_name="s"),
        compiler_params=pltpu.CompilerParams(use_tc_tiling_on_sc=False),
        scratch_types=[
            pltpu.VMEM((CH, K), jnp.int32),
            pltpu.VMEM((CH, K), jnp.int32),
            pltpu.VMEM((CH, K), jnp.float32),
            pltpu.VMEM((2, K, H // 2), jnp.int32),
            pltpu.VMEM((K, H), jnp.float32),
            pltpu.VMEM_SHARED((N, H), jnp.float32),
            pltpu.SemaphoreType.DMA,
        ],
    )


def _agg(y_bf16, src, dst, w):
    y_packed = lax.bitcast_convert_type(
        y_bf16.reshape(N, H // 2, 2), jnp.int32)
    return _make_agg()(y_packed, src, dst, w)


_COL_ORDER = tuple(
    32 * j + 2 * t + half
    for j in range(H // 32) for half in range(2) for t in range(16))


def _mm_body(x_ref, w_ref, o_ref):
    o_ref[...] = jnp.dot(x_ref[...], w_ref[...],
                         preferred_element_type=jnp.float32
                         ).astype(jnp.bfloat16)


def _bn_mm_body(p_ref, g_ref, be_ref, w_ref, o_ref):
    a = p_ref[0] + p_ref[1]
    mu = jnp.mean(a, axis=0, keepdims=True)
    var = jnp.mean((a - mu) * (a - mu), axis=0, keepdims=True)
    h = (a - mu) * lax.rsqrt(var + 1e-5) * g_ref[...] + be_ref[...]
    h = jnp.maximum(h, 0.0)
    o_ref[...] = jnp.dot(h, w_ref[...], preferred_element_type=jnp.float32
                         ).astype(jnp.bfloat16)


def _final_body(p_ref, g_ref, be_ref, wc_ref, bc_ref, o_ref):
    a = p_ref[0] + p_ref[1]
    mu = jnp.mean(a, axis=0, keepdims=True)
    var = jnp.mean((a - mu) * (a - mu), axis=0, keepdims=True)
    h = (a - mu) * lax.rsqrt(var + 1e-5) * g_ref[...] + be_ref[...]
    h = jnp.maximum(h, 0.0)
    logits = jnp.dot(h, wc_ref[...], preferred_element_type=jnp.float32)
    logits = logits + bc_ref[...]
    m = jnp.max(logits, axis=1, keepdims=True)
    ex = jnp.exp(logits - m)
    lse = jnp.log(jnp.sum(ex, axis=1, keepdims=True)) + m
    o_ref[...] = logits - lse


def _tc_call(body, out_shape, *args, dtype=jnp.float32):
    return pl.pallas_call(
        body, out_shape=jax.ShapeDtypeStruct(out_shape, dtype))(*args)


@jax.jit
def kernel(x, edge_index, edge_weight, W0, b0, g0, be0, W1, b1, g1, be1,
           Wc, bc):
    del b0, b1
    pad = E_PAD - E
    src = jnp.concatenate([edge_index[0], jnp.zeros((pad,), jnp.int32)])
    dst = jnp.concatenate([edge_index[1], jnp.zeros((pad,), jnp.int32)])
    w = jnp.concatenate([edge_weight, jnp.zeros((pad,), jnp.float32)])
    src = src.reshape(TOT_BLOCKS, K)
    dst = dst.reshape(TOT_BLOCKS, K)
    w = w.reshape(TOT_BLOCKS, K)

    order = jnp.asarray(_COL_ORDER, dtype=jnp.int32)
    g0p = g0[order].reshape(1, H)
    be0p = be0[order].reshape(1, H)
    W1p = W1[order, :]
    g1p = g1[order].reshape(1, H)
    be1p = be1[order].reshape(1, H)
    Wcp = Wc[order, :]

    y0 = _tc_call(_mm_body, (N, H), x, W0, dtype=jnp.bfloat16)
    p = _agg(y0, src, dst, w)
    y1 = _tc_call(_bn_mm_body, (N, H), p, g0p, be0p, W1p,
                  dtype=jnp.bfloat16)
    q = _agg(y1, src, dst, w)
    return _tc_call(_final_body, (N, C), q, g1p, be1p, Wcp,
                    bc.reshape(1, C))


# speedup vs baseline: 1.3651x; 1.3651x over previous
import functools

import jax
import jax.numpy as jnp
from jax import lax
from jax.experimental import pallas as pl
from jax.experimental.pallas import tpu as pltpu
from jax.experimental.pallas import tpu_sc as plsc

N, E, D, H, C = 10000, 320000, 128, 128, 40

NC = 2
NS = 16
K = 128
B_CORE = (144, 16)
TOT_BLOCKS = NS * (B_CORE[0] + B_CORE[1])
E_PAD = TOT_BLOCKS * K
CH = 16
MAX_CHUNKS = B_CORE[0] // CH
ROWS_PER_TILE = 624


def _splat(wvec, l):
    return lax.gather(
        wvec, jnp.full((16, 1), l, jnp.int32),
        lax.GatherDimensionNumbers(
            offset_dims=(), collapsed_slice_dims=(0,), start_index_map=(0,)),
        slice_sizes=(1,),
        mode=lax.GatherScatterMode.PROMISE_IN_BOUNDS)


def _agg_body(y_hbm, src_hbm, dst_hbm, w_hbm, out_hbm,
              src_v, dst_v, w_v, rows_v, acc_sh, sem_g, sem_s):
    c = lax.axis_index("c")
    s = lax.axis_index("s")
    gbase = jnp.where(c == 0, s * B_CORE[0],
                      NS * B_CORE[0] + s * B_CORE[1])
    nch = jnp.where(c == 0, B_CORE[0] // CH, B_CORE[1] // CH)

    def zero_body(i, _):
        r = i // 8
        j = i % 8
        rows_v[0, r, pl.ds(j * 16, 16)] = jnp.zeros((16,), jnp.float32)
        return 0
    lax.fori_loop(0, K * 8, zero_body, 0)
    base_r = s * ROWS_PER_TILE
    for off in (0, 128, 256, 384, ROWS_PER_TILE - K):
        pltpu.sync_copy(rows_v.at[0], acc_sh.at[pl.ds(base_r + off, K)])

    @pl.when(s == NS - 1)
    def _zero_tail():
        pltpu.sync_copy(rows_v.at[0], acc_sh.at[pl.ds(N - K, K)])
    plsc.subcore_barrier()

    def start_gather(i, buf):
        pltpu.async_copy(y_hbm.at[src_v.at[i]], rows_v.at[buf], sem_g)

    def wait_gather(buf):
        pltpu.make_async_copy(y_hbm.at[pl.ds(0, K)], rows_v.at[buf],
                              sem_g).wait()

    def start_scatter(i, buf):
        pltpu.async_copy(rows_v.at[buf], acc_sh.at[dst_v.at[i]], sem_s,
                         add=True)

    def wait_scatter(buf):
        pltpu.make_async_copy(rows_v.at[buf], acc_sh.at[pl.ds(0, K)],
                              sem_s).wait()

    def scale(i, buf):
        def scale_body(g, _):
            wvec = w_v[i, pl.ds(g * 16, 16)]
            for l in range(16):
                k = g * 16 + l
                wk = _splat(wvec, l)
                for j in range(8):
                    sl = pl.ds(j * 16, 16)
                    rows_v[buf, k, sl] = rows_v[buf, k, sl] * wk
            return 0
        lax.fori_loop(0, K // 16, scale_body, 0)

    def chunk_body(ch, _):
        @pl.when(ch < nch)
        def _do_chunk():
            row0 = gbase + ch * CH
            pltpu.sync_copy(src_hbm.at[pl.ds(row0, CH)], src_v)
            pltpu.sync_copy(dst_hbm.at[pl.ds(row0, CH)], dst_v)
            pltpu.sync_copy(w_hbm.at[pl.ds(row0, CH)], w_v)
            start_gather(0, 0)

            def block_body(g, _):
                for b in range(2):
                    i = g * 2 + b
                    wait_gather(b)

                    @pl.when(i >= 1)
                    def _drain_prev():
                        wait_scatter(1 - b)

                    @pl.when(i + 1 < CH)
                    def _next():
                        start_gather(i + 1, 1 - b)
                    scale(i, b)
                    start_scatter(i, b)
                return 0
            lax.fori_loop(0, CH // 2, block_body, 0)
            wait_scatter(1)
        return 0
    lax.fori_loop(0, MAX_CHUNKS, chunk_body, 0)

    plsc.subcore_barrier()
    pltpu.sync_copy(acc_sh.at[pl.ds(base_r, ROWS_PER_TILE)],
                    out_hbm.at[c].at[pl.ds(base_r, ROWS_PER_TILE)])

    @pl.when(s == NS - 1)
    def _copy_tail():
        tail = N - NS * ROWS_PER_TILE
        pltpu.sync_copy(acc_sh.at[pl.ds(NS * ROWS_PER_TILE, tail)],
                        out_hbm.at[c].at[pl.ds(NS * ROWS_PER_TILE, tail)])


@functools.lru_cache(maxsize=1)
def _make_agg():
    return pl.kernel(
        _agg_body,
        out_type=jax.ShapeDtypeStruct((NC, N, H), jnp.float32),
        mesh=plsc.VectorSubcoreMesh(core_axis_name="c", subcore_axis_name="s"),
        scratch_types=[
            pltpu.VMEM((CH, K), jnp.int32),
            pltpu.VMEM((CH, K), jnp.int32),
            pltpu.VMEM((CH, K), jnp.float32),
            pltpu.VMEM((2, K, H), jnp.float32),
            pltpu.VMEM_SHARED((N, H), jnp.float32),
            pltpu.SemaphoreType.DMA,
            pltpu.SemaphoreType.DMA,
        ],
    )


def _agg(y, src, dst, w):
    return _make_agg()(y, src, dst, w)


def _mm_body(x_ref, w_ref, o_ref):
    o_ref[...] = jnp.dot(x_ref[...], w_ref[...],
                         preferred_element_type=jnp.float32)


def _bn_mm_body(p_ref, g_ref, be_ref, w_ref, o_ref):
    a = p_ref[0] + p_ref[1]
    mu = jnp.mean(a, axis=0, keepdims=True)
    var = jnp.mean((a - mu) * (a - mu), axis=0, keepdims=True)
    h = (a - mu) * lax.rsqrt(var + 1e-5) * g_ref[...] + be_ref[...]
    h = jnp.maximum(h, 0.0)
    o_ref[...] = jnp.dot(h, w_ref[...], preferred_element_type=jnp.float32)


def _final_body(p_ref, g_ref, be_ref, wc_ref, bc_ref, o_ref):
    a = p_ref[0] + p_ref[1]
    mu = jnp.mean(a, axis=0, keepdims=True)
    var = jnp.mean((a - mu) * (a - mu), axis=0, keepdims=True)
    h = (a - mu) * lax.rsqrt(var + 1e-5) * g_ref[...] + be_ref[...]
    h = jnp.maximum(h, 0.0)
    logits = jnp.dot(h, wc_ref[...], preferred_element_type=jnp.float32)
    logits = logits + bc_ref[...]
    m = jnp.max(logits, axis=1, keepdims=True)
    ex = jnp.exp(logits - m)
    lse = jnp.log(jnp.sum(ex, axis=1, keepdims=True)) + m
    o_ref[...] = logits - lse


def _tc_call(body, out_shape, *args, dtype=jnp.float32):
    return pl.pallas_call(
        body, out_shape=jax.ShapeDtypeStruct(out_shape, dtype))(*args)


@jax.jit
def kernel(x, edge_index, edge_weight, W0, b0, g0, be0, W1, b1, g1, be1,
           Wc, bc):
    del b0, b1
    pad = E_PAD - E
    src = jnp.concatenate([edge_index[0], jnp.zeros((pad,), jnp.int32)])
    dst = jnp.concatenate([edge_index[1], jnp.zeros((pad,), jnp.int32)])
    w = jnp.concatenate([edge_weight, jnp.zeros((pad,), jnp.float32)])
    src = src.reshape(TOT_BLOCKS, K)
    dst = dst.reshape(TOT_BLOCKS, K)
    w = w.reshape(TOT_BLOCKS, K)

    y0 = _tc_call(_mm_body, (N, H), x, W0)
    p = _agg(y0, src, dst, w)
    y1 = _tc_call(_bn_mm_body, (N, H), p, g0.reshape(1, H),
                  be0.reshape(1, H), W1)
    q = _agg(y1, src, dst, w)
    return _tc_call(_final_body, (N, C), q, g1.reshape(1, H),
                    be1.reshape(1, H), Wc, bc.reshape(1, C))
